# Initial kernel scaffold; baseline (speedup 1.0000x reference)
import jax
import jax.numpy as jnp
from jax.experimental import pallas as pl


def kernel(text, emb, fc_w, fc_b):
    raise NotImplementedError("write your pallas kernel here")


# speedup vs baseline: 2.6471x; 2.6471x over previous
import jax
import jax.numpy as jnp
from jax import lax
from jax.experimental import pallas as pl
from jax.experimental.pallas import tpu as pltpu, tpu_sc as plsc

VOCAB = 1000000
EMBED_DIM = 32
NUM_CLASS = 4
BATCH = 16384
SEQ = 50

_info = plsc.get_sparse_core_info()
_NC, _NS, _L = _info.num_cores, _info.num_subcores, _info.num_lanes
_NW = _NC * _NS
_RPW = BATCH // _NW
_G = 16
_NGRP = _RPW // _G


_NBUF = 8


def _body(text_hbm, emb_hbm, wbc_hbm, bbc_hbm, out_hbm,
          idx_all, bufs, wbc_v, bbc_v, sums, outbuf, sems):
    wid = lax.axis_index("s") * _NC + lax.axis_index("c")
    base = wid * _RPW

    pltpu.sync_copy(text_hbm.at[pl.ds(base, _RPW), :], idx_all)
    pltpu.sync_copy(wbc_hbm, wbc_v)
    pltpu.sync_copy(bbc_hbm, bbc_v)

    inv = jnp.float32(1.0 / SEQ)
    lanes = lax.iota(jnp.int32, _L)
    lanes32 = lanes * EMBED_DIM
    lanes4 = lanes * NUM_CLASS

    def gather(r, b):
        return pltpu.async_copy(emb_hbm.at[idx_all.at[r]], bufs.at[b], sems.at[b])

    for b in range(_NBUF):
        gather(b, b)

    def step(g, _):
        r0 = g * _G
        for j in range(_G):
            r = r0 + j
            b = j % _NBUF
            buf = bufs.at[b]
            pltpu.make_async_copy(emb_hbm.at[idx_all.at[r]], buf,
                                  sems.at[b]).wait()

            s0 = buf[0, pl.ds(0, _L)]
            s1 = buf[0, pl.ds(_L, _L)]
            for i in range(1, SEQ):
                s0 = s0 + buf[i, pl.ds(0, _L)]
                s1 = s1 + buf[i, pl.ds(_L, _L)]
            sums[pl.ds(j * EMBED_DIM, _L)] = s0 * inv
            sums[pl.ds(j * EMBED_DIM + _L, _L)] = s1 * inv

            @pl.when(r + _NBUF < _RPW)
            def _():
                gather(r + _NBUF, b)

        acc = [bbc_v[c, :] for c in range(NUM_CLASS)]
        for d in range(EMBED_DIM):
            col = plsc.load_gather(sums, [lanes32 + d])
            for c in range(NUM_CLASS):
                acc[c] = acc[c] + col * wbc_v[c, d, :]
        gbase = g * (_G * NUM_CLASS)
        for c in range(NUM_CLASS):
            plsc.store_scatter(outbuf, [gbase + lanes4 + c], acc[c])
        return 0

    lax.fori_loop(0, _NGRP, step, 0)

    pltpu.sync_copy(outbuf, out_hbm.at[pl.ds(wid * (_RPW * NUM_CLASS),
                                             _RPW * NUM_CLASS)])


@jax.jit
def _run(text, emb, fc_w, fc_b):
    mesh = plsc.VectorSubcoreMesh(core_axis_name="c", subcore_axis_name="s")
    f = pl.kernel(
        _body,
        out_type=jax.ShapeDtypeStruct((BATCH * NUM_CLASS,), jnp.float32),
        mesh=mesh,
        compiler_params=pltpu.CompilerParams(needs_layout_passes=False,
                                             use_tc_tiling_on_sc=False),
        scratch_types=[
            pltpu.VMEM((_RPW, SEQ), jnp.int32),
            pltpu.VMEM((_NBUF, SEQ, EMBED_DIM), jnp.float32),
            pltpu.VMEM((NUM_CLASS, EMBED_DIM, _L), jnp.float32),
            pltpu.VMEM((NUM_CLASS, _L), jnp.float32),
            pltpu.VMEM((_G * EMBED_DIM,), jnp.float32),
            pltpu.VMEM((_RPW * NUM_CLASS,), jnp.float32),
            pltpu.SemaphoreType.DMA((_NBUF,)),
        ],
    )
    wbc = jnp.broadcast_to(fc_w[:, :, None], (NUM_CLASS, EMBED_DIM, _L))
    bbc = jnp.broadcast_to(fc_b[:, None], (NUM_CLASS, _L))
    out = f(text.astype(jnp.int32), emb, wbc, bbc)
    return out.reshape(BATCH, NUM_CLASS)


def kernel(text, emb, fc_w, fc_b):
    return _run(text, emb, fc_w, fc_b)


# speedup vs baseline: 2.6777x; 1.0116x over previous
import jax
import jax.numpy as jnp
from jax import lax
from jax.experimental import pallas as pl
from jax.experimental.pallas import tpu as pltpu, tpu_sc as plsc

VOCAB = 1000000
EMBED_DIM = 32
NUM_CLASS = 4
BATCH = 16384
SEQ = 50

_info = plsc.get_sparse_core_info()
_NC, _NS, _L = _info.num_cores, _info.num_subcores, _info.num_lanes
_NW = _NC * _NS
_RPW = BATCH // _NW
_G = 16
_NGRP = _RPW // _G
_GTOK = _G * SEQ
_NBUF = 3


def _body(text_hbm, emb_hbm, wbc_hbm, bbc_hbm, out_hbm,
          idx_all, bufs, wbc_v, bbc_v, sums, outbuf, sems):
    wid = lax.axis_index("s") * _NC + lax.axis_index("c")

    pltpu.sync_copy(text_hbm.at[pl.ds(wid * _NGRP, _NGRP), :], idx_all)
    pltpu.sync_copy(wbc_hbm, wbc_v)
    pltpu.sync_copy(bbc_hbm, bbc_v)

    inv = jnp.float32(1.0 / SEQ)
    lanes = lax.iota(jnp.int32, _L)
    lanes32 = lanes * EMBED_DIM
    lanes4 = lanes * NUM_CLASS

    def gather(g, b):
        return pltpu.async_copy(emb_hbm.at[idx_all.at[g]], bufs.at[b],
                                sems.at[b])

    for b in range(_NBUF):
        gather(b, b)

    def step(g, _):
        b = lax.rem(g, _NBUF)
        buf = bufs.at[b]
        pltpu.make_async_copy(emb_hbm.at[idx_all.at[g]], buf,
                              sems.at[b]).wait()

        for j in range(_G):
            t0 = j * SEQ
            s0 = buf[t0, pl.ds(0, _L)]
            s1 = buf[t0, pl.ds(_L, _L)]
            for i in range(1, SEQ):
                s0 = s0 + buf[t0 + i, pl.ds(0, _L)]
                s1 = s1 + buf[t0 + i, pl.ds(_L, _L)]
            sums[pl.ds(j * EMBED_DIM, _L)] = s0 * inv
            sums[pl.ds(j * EMBED_DIM + _L, _L)] = s1 * inv

        @pl.when(g + _NBUF < _NGRP)
        def _():
            gather(g + _NBUF, b)

        acc = [bbc_v[c, :] for c in range(NUM_CLASS)]
        for d in range(EMBED_DIM):
            col = plsc.load_gather(sums, [lanes32 + d])
            for c in range(NUM_CLASS):
                acc[c] = acc[c] + col * wbc_v[c, d, :]
        gbase = g * (_G * NUM_CLASS)
        for c in range(NUM_CLASS):
            plsc.store_scatter(outbuf, [gbase + lanes4 + c], acc[c])
        return 0

    lax.fori_loop(0, _NGRP, step, 0)

    pltpu.sync_copy(outbuf, out_hbm.at[pl.ds(wid * (_RPW * NUM_CLASS),
                                             _RPW * NUM_CLASS)])


@jax.jit
def _run(text, emb, fc_w, fc_b):
    mesh = plsc.VectorSubcoreMesh(core_axis_name="c", subcore_axis_name="s")
    f = pl.kernel(
        _body,
        out_type=jax.ShapeDtypeStruct((BATCH * NUM_CLASS,), jnp.float32),
        mesh=mesh,
        compiler_params=pltpu.CompilerParams(needs_layout_passes=False,
                                             use_tc_tiling_on_sc=False),
        scratch_types=[
            pltpu.VMEM((_NGRP, _GTOK), jnp.int32),
            pltpu.VMEM((_NBUF, _GTOK, EMBED_DIM), jnp.float32),
            pltpu.VMEM((NUM_CLASS, EMBED_DIM, _L), jnp.float32),
            pltpu.VMEM((NUM_CLASS, _L), jnp.float32),
            pltpu.VMEM((_G * EMBED_DIM,), jnp.float32),
            pltpu.VMEM((_RPW * NUM_CLASS,), jnp.float32),
            pltpu.SemaphoreType.DMA((_NBUF,)),
        ],
    )
    wbc = jnp.broadcast_to(fc_w[:, :, None], (NUM_CLASS, EMBED_DIM, _L))
    bbc = jnp.broadcast_to(fc_b[:, None], (NUM_CLASS, _L))
    text2 = text.astype(jnp.int32).reshape(BATCH * SEQ // _GTOK, _GTOK)
    out = f(text2, emb, wbc, bbc)
    return out.reshape(BATCH, NUM_CLASS)


def kernel(text, emb, fc_w, fc_b):
    return _run(text, emb, fc_w, fc_b)
